# baseline (speedup 1.0000x reference)
import functools

import jax
import jax.numpy as jnp
from jax import lax
from jax.experimental import pallas as pl
from jax.experimental.pallas import tpu as pltpu
from jax.experimental.pallas import tpu_sc as plsc

EMB = 512
HID = 1024
EPS = 1e-07

_NC = 2
_NS = 16
_NW = _NC * _NS
_K = 4
_SUB = 32
_GRP = EMB // 32


_MASK_LO = jnp.int32(0xFFFF)
_MASK_HI = jnp.int32(-65536)
_RND = jnp.int32(0x7FFF)


def _bf16_bits(v):
    lsb = (v >> 16) & jnp.int32(1)
    return ((v + _RND + lsb) >> 16) & _MASK_LO


def _pack_rows(buf_v, bbuf_v):
    def row(r, _):
        for g in range(_GRP):
            cb = g * 32
            a = buf_v[r, pl.ds(cb, 16)]
            b = buf_v[r, pl.ds(cb + 16, 16)]
            lsb_b = (b >> 16) & jnp.int32(1)
            hi_b = (b + _RND + lsb_b) & _MASK_HI
            bbuf_v[pl.ds(r * (EMB // 2) + g * 16, 16)] = _bf16_bits(a) | hi_b
        return _

    lax.fori_loop(0, _SUB, row, None)


def _sc_gather_body(ids_hbm, table_hbm, out_hbm, idx_v, buf0, buf1, bb0, bb1,
                    g0, g1, s0, s1, *, k, sch, seq_len, bsz):
    tok_per_w = (sch * bsz) // _NW
    n_sub = tok_per_w // _SUB
    wpb = _NW // bsz
    wid = lax.axis_index("s") * _NC + lax.axis_index("c")
    myb = wid // wpb
    myj = wid % wpb
    col = k * sch + myj * tok_per_w
    pltpu.sync_copy(ids_hbm.at[myb, pl.ds(col, tok_per_w)], idx_v)
    bufs = (buf0, buf1)
    bbufs = (bb0, bb1)
    gsems = (g0, g1)
    ssems = (s0, s1)
    gathers = [None, None]
    scatters = [None, None]
    gathers[0] = pltpu.async_copy(
        table_hbm.at[idx_v.at[pl.ds(0, _SUB)]], bufs[0], gsems[0])
    base_out = wid * tok_per_w
    for c in range(n_sub):
        cur = c % 2
        nxt = (c + 1) % 2
        if c + 1 < n_sub:
            gathers[nxt] = pltpu.async_copy(
                table_hbm.at[idx_v.at[pl.ds((c + 1) * _SUB, _SUB)]],
                bufs[nxt], gsems[nxt])
        gathers[cur].wait()
        if scatters[cur] is not None:
            scatters[cur].wait()
        _pack_rows(bufs[cur], bbufs[cur])
        scatters[cur] = pltpu.async_copy(
            bbufs[cur],
            out_hbm.at[pl.ds((base_out + c * _SUB) * (EMB // 2),
                             _SUB * EMB // 2)],
            ssems[cur])
    for sc in scatters:
        if sc is not None:
            sc.wait()


def _sc_gather_chunk(ids2d, word_table, k, sch, seq_len, bsz):
    tok_per_w = (sch * bsz) // _NW
    mesh = plsc.VectorSubcoreMesh(core_axis_name="c", subcore_axis_name="s")
    body = functools.partial(_sc_gather_body, k=k, sch=sch, seq_len=seq_len,
                             bsz=bsz)
    kern = functools.partial(
        pl.kernel,
        mesh=mesh,
        out_type=jax.ShapeDtypeStruct((sch * bsz * EMB // 2,), jnp.int32),
        scratch_types=[
            pltpu.VMEM((tok_per_w,), jnp.int32),
            pltpu.VMEM((_SUB, EMB), jnp.int32),
            pltpu.VMEM((_SUB, EMB), jnp.int32),
            pltpu.VMEM((_SUB * EMB // 2,), jnp.int32),
            pltpu.VMEM((_SUB * EMB // 2,), jnp.int32),
            pltpu.SemaphoreType.DMA,
            pltpu.SemaphoreType.DMA,
            pltpu.SemaphoreType.DMA,
            pltpu.SemaphoreType.DMA,
        ],
    )(body)
    packed = kern(ids2d, word_table)
    return lax.bitcast_convert_type(packed, jnp.bfloat16).reshape(
        sch * bsz, EMB)


def _tc_body_first(x_ref, pos_ref, w_ref, g_ref, b_ref, o_ref):
    x = (x_ref[...].astype(jnp.float32)
         + pos_ref[...].astype(jnp.float32)).astype(jnp.bfloat16)
    h = jnp.dot(x, w_ref[...], preferred_element_type=jnp.float32)
    mu = jnp.mean(h, axis=-1, keepdims=True)
    var = jnp.mean((h - mu) ** 2, axis=-1, keepdims=True)
    o_ref[...] = (h - mu) * lax.rsqrt(var + EPS) * g_ref[...] + b_ref[...]


def _tc_body_chained(prev_ref, x_ref, pos_ref, w_ref, g_ref, b_ref, o_ref):
    del prev_ref
    _tc_body_first(x_ref, pos_ref, w_ref, g_ref, b_ref, o_ref)


def _tc_chunk(k, n_tok, nbatch, sch, embeds_k, pos_table, wb, g2, b2, prev):
    blk = sch
    blocks_per_batch = (n_tok // nbatch) // blk
    x_spec = pl.BlockSpec((blk, EMB), lambda j, b: (b, 0))
    pos_spec = pl.BlockSpec((blk, EMB), lambda j, b: (k, 0))
    w_spec = pl.BlockSpec((EMB, HID), lambda j, b: (0, 0))
    v_spec = pl.BlockSpec((1, HID), lambda j, b: (0, 0))
    out_spec = pl.BlockSpec(
        (blk, HID), lambda j, b: (b * blocks_per_batch + k, 0))
    out_shape = jax.ShapeDtypeStruct((n_tok, HID), jnp.float32)
    grid = (1, nbatch)
    if prev is None:
        return pl.pallas_call(
            _tc_body_first,
            grid=grid,
            in_specs=[x_spec, pos_spec, w_spec, v_spec, v_spec],
            out_specs=out_spec,
            out_shape=out_shape,
        )(embeds_k, pos_table, wb, g2, b2)
    return pl.pallas_call(
        _tc_body_chained,
        grid=grid,
        in_specs=[pl.BlockSpec(memory_space=pl.MemorySpace.ANY),
                  x_spec, pos_spec, w_spec, v_spec, v_spec],
        out_specs=out_spec,
        out_shape=out_shape,
        input_output_aliases={0: 0},
    )(prev, embeds_k, pos_table, wb, g2, b2)


def kernel(input_ids, word_table, pos_table, proj_w, ln_gamma, ln_beta):
    bsz, seq_len = input_ids.shape
    n_tok = bsz * seq_len
    ids2d = input_ids.astype(jnp.int32)
    sch = seq_len // _K
    wb = (proj_w.reshape(EMB // 32, 2, 16, HID).transpose(0, 2, 1, 3)
          .reshape(EMB, HID).astype(jnp.bfloat16))
    posp = (pos_table.reshape(seq_len, EMB // 32, 2, 16)
            .transpose(0, 1, 3, 2).reshape(seq_len, EMB)
            .astype(jnp.bfloat16))
    g2 = ln_gamma.reshape(1, HID)
    b2 = ln_beta.reshape(1, HID)
    wt_i32 = lax.bitcast_convert_type(word_table, jnp.int32)
    embeds = [
        _sc_gather_chunk(ids2d, wt_i32, k, sch, seq_len, bsz)
        for k in range(_K)
    ]
    o = None
    for k in range(_K):
        o = _tc_chunk(k, n_tok, bsz, sch, embeds[k], posp, wb, g2, b2, o)
    return o.reshape(bsz, seq_len, HID)


# speedup vs baseline: 4.7349x; 4.7349x over previous
import functools

import jax
import jax.numpy as jnp
from jax import lax
from jax.experimental import pallas as pl
from jax.experimental.pallas import tpu as pltpu
from jax.experimental.pallas import tpu_sc as plsc

EMB = 512
HID = 1024
EPS = 1e-07

_NC = 2
_NS = 16
_NW = _NC * _NS
_CHUNKS = ((0, 256), (256, 512), (768, 512), (1280, 768))
_BLK = 256


def _sc_gather_body(ids_hbm, table_hbm, out_hbm, idx_v, buf_v, sem, *, start,
                    sch, seq_len, bsz):
    tok_per_w = (sch * bsz) // _NW
    wpb = _NW // bsz
    wid = lax.axis_index("s") * _NC + lax.axis_index("c")
    myb = wid // wpb
    myj = wid % wpb
    base_in = myb * seq_len + start + myj * tok_per_w
    pltpu.sync_copy(ids_hbm.at[pl.ds(base_in, tok_per_w)], idx_v)
    pltpu.async_copy(table_hbm.at[idx_v], buf_v, sem).wait()
    pltpu.sync_copy(buf_v, out_hbm.at[pl.ds(wid * tok_per_w, tok_per_w)])


def _sc_gather_chunk(ids2d, word_table, start, sch, seq_len, bsz):
    tok_per_w = (sch * bsz) // _NW
    mesh = plsc.VectorSubcoreMesh(core_axis_name="c", subcore_axis_name="s")
    body = functools.partial(_sc_gather_body, start=start, sch=sch,
                             seq_len=seq_len, bsz=bsz)
    kern = functools.partial(
        pl.kernel,
        mesh=mesh,
        out_type=jax.ShapeDtypeStruct((sch * bsz, EMB), jnp.float32),
        scratch_types=[
            pltpu.VMEM((tok_per_w,), jnp.int32),
            pltpu.VMEM((tok_per_w, EMB), jnp.float32),
            pltpu.SemaphoreType.DMA,
        ],
    )(body)
    return kern(ids2d, word_table)


def _tc_body_first(x_ref, pos_ref, w_ref, g_ref, b_ref, o_ref):
    x = (x_ref[...] + pos_ref[...]).astype(jnp.bfloat16)
    h = jnp.dot(x, w_ref[...], preferred_element_type=jnp.float32)
    mu = jnp.mean(h, axis=-1, keepdims=True)
    var = jnp.mean((h - mu) ** 2, axis=-1, keepdims=True)
    o_ref[...] = (h - mu) * lax.rsqrt(var + EPS) * g_ref[...] + b_ref[...]


def _tc_body_chained(prev_ref, x_ref, pos_ref, w_ref, g_ref, b_ref, o_ref):
    del prev_ref
    _tc_body_first(x_ref, pos_ref, w_ref, g_ref, b_ref, o_ref)


def _tc_chunk(start, sch, n_tok, nbatch, embeds_k, pos_table, wb, g2, b2,
              prev):
    blk = _BLK
    jb = sch // blk
    pos0 = start // blk
    blocks_per_batch = (n_tok // nbatch) // blk
    x_spec = pl.BlockSpec((blk, EMB), lambda j, b: (b * jb + j, 0))
    pos_spec = pl.BlockSpec((blk, EMB), lambda j, b: (pos0 + j, 0))
    w_spec = pl.BlockSpec((EMB, HID), lambda j, b: (0, 0))
    v_spec = pl.BlockSpec((1, HID), lambda j, b: (0, 0))
    out_spec = pl.BlockSpec(
        (blk, HID), lambda j, b: (b * blocks_per_batch + pos0 + j, 0))
    out_shape = jax.ShapeDtypeStruct((n_tok, HID), jnp.float32)
    grid = (jb, nbatch)
    if prev is None:
        return pl.pallas_call(
            _tc_body_first,
            grid=grid,
            in_specs=[x_spec, pos_spec, w_spec, v_spec, v_spec],
            out_specs=out_spec,
            out_shape=out_shape,
        )(embeds_k, pos_table, wb, g2, b2)
    return pl.pallas_call(
        _tc_body_chained,
        grid=grid,
        in_specs=[pl.BlockSpec(memory_space=pl.MemorySpace.ANY),
                  x_spec, pos_spec, w_spec, v_spec, v_spec],
        out_specs=out_spec,
        out_shape=out_shape,
        input_output_aliases={0: 0},
    )(prev, embeds_k, pos_table, wb, g2, b2)


def kernel(input_ids, word_table, pos_table, proj_w, ln_gamma, ln_beta):
    bsz, seq_len = input_ids.shape
    n_tok = bsz * seq_len
    ids_flat = input_ids.reshape(-1).astype(jnp.int32)
    wb = proj_w.astype(jnp.bfloat16)
    g2 = ln_gamma.reshape(1, HID)
    b2 = ln_beta.reshape(1, HID)
    embeds = [
        _sc_gather_chunk(ids_flat, word_table, start, sch, seq_len, bsz)
        for start, sch in _CHUNKS
    ]
    o = None
    for (start, sch), e in zip(_CHUNKS, embeds):
        o = _tc_chunk(start, sch, n_tok, bsz, e, pos_table, wb, g2, b2, o)
    return o.reshape(bsz, seq_len, HID)


# speedup vs baseline: 5.0380x; 1.0640x over previous
import functools

import jax
import jax.numpy as jnp
from jax import lax
from jax.experimental import pallas as pl
from jax.experimental.pallas import tpu as pltpu
from jax.experimental.pallas import tpu_sc as plsc

EMB = 512
HID = 1024
EPS = 1e-07

_NC = 2
_NS = 16
_NW = _NC * _NS
_CHUNKS = ((0, 256), (256, 256), (512, 512), (1024, 512), (1536, 512))


def _sc_gather_body(ids_hbm, table_hbm, out_hbm, idx_v, buf_v, sem, *, start,
                    sch, seq_len, bsz):
    tok_per_w = (sch * bsz) // _NW
    wpb = _NW // bsz
    wid = lax.axis_index("s") * _NC + lax.axis_index("c")
    myb = wid // wpb
    myj = wid % wpb
    base_in = myb * seq_len + start + myj * tok_per_w
    pltpu.sync_copy(ids_hbm.at[pl.ds(base_in, tok_per_w)], idx_v)
    pltpu.async_copy(table_hbm.at[idx_v], buf_v, sem).wait()
    pltpu.sync_copy(buf_v, out_hbm.at[pl.ds(wid * tok_per_w, tok_per_w)])


def _sc_gather_chunk(ids2d, word_table, start, sch, seq_len, bsz):
    tok_per_w = (sch * bsz) // _NW
    mesh = plsc.VectorSubcoreMesh(core_axis_name="c", subcore_axis_name="s")
    body = functools.partial(_sc_gather_body, start=start, sch=sch,
                             seq_len=seq_len, bsz=bsz)
    kern = functools.partial(
        pl.kernel,
        mesh=mesh,
        out_type=jax.ShapeDtypeStruct((sch * bsz, EMB), jnp.float32),
        scratch_types=[
            pltpu.VMEM((tok_per_w,), jnp.int32),
            pltpu.VMEM((tok_per_w, EMB), jnp.float32),
            pltpu.SemaphoreType.DMA,
        ],
    )(body)
    return kern(ids2d, word_table)


def _tc_body_first(x_ref, pos_ref, w_ref, g_ref, b_ref, o_ref):
    x = (x_ref[...] + pos_ref[...]).astype(jnp.bfloat16)
    h = jnp.dot(x, w_ref[...], preferred_element_type=jnp.float32)
    mu = jnp.mean(h, axis=-1, keepdims=True)
    var = jnp.mean((h - mu) ** 2, axis=-1, keepdims=True)
    o_ref[...] = (h - mu) * lax.rsqrt(var + EPS) * g_ref[...] + b_ref[...]


def _tc_body_chained(prev_ref, x_ref, pos_ref, w_ref, g_ref, b_ref, o_ref):
    del prev_ref
    _tc_body_first(x_ref, pos_ref, w_ref, g_ref, b_ref, o_ref)


def _tc_chunk(start, sch, n_tok, nbatch, embeds_k, pos_table, wb, g2, b2,
              prev):
    blk = sch if start % sch == 0 else 256
    jb = sch // blk
    pos0 = start // blk
    blocks_per_batch = (n_tok // nbatch) // blk
    x_spec = pl.BlockSpec((blk, EMB), lambda j, b: (b * jb + j, 0))
    pos_spec = pl.BlockSpec((blk, EMB), lambda j, b: (pos0 + j, 0))
    w_spec = pl.BlockSpec((EMB, HID), lambda j, b: (0, 0))
    v_spec = pl.BlockSpec((1, HID), lambda j, b: (0, 0))
    out_spec = pl.BlockSpec(
        (blk, HID), lambda j, b: (b * blocks_per_batch + pos0 + j, 0))
    out_shape = jax.ShapeDtypeStruct((n_tok, HID), jnp.float32)
    grid = (jb, nbatch)
    if prev is None:
        return pl.pallas_call(
            _tc_body_first,
            grid=grid,
            in_specs=[x_spec, pos_spec, w_spec, v_spec, v_spec],
            out_specs=out_spec,
            out_shape=out_shape,
        )(embeds_k, pos_table, wb, g2, b2)
    return pl.pallas_call(
        _tc_body_chained,
        grid=grid,
        in_specs=[pl.BlockSpec(memory_space=pl.MemorySpace.ANY),
                  x_spec, pos_spec, w_spec, v_spec, v_spec],
        out_specs=out_spec,
        out_shape=out_shape,
        input_output_aliases={0: 0},
    )(prev, embeds_k, pos_table, wb, g2, b2)


def kernel(input_ids, word_table, pos_table, proj_w, ln_gamma, ln_beta):
    bsz, seq_len = input_ids.shape
    n_tok = bsz * seq_len
    ids_flat = input_ids.reshape(-1).astype(jnp.int32)
    wb = proj_w.astype(jnp.bfloat16)
    g2 = ln_gamma.reshape(1, HID)
    b2 = ln_beta.reshape(1, HID)
    embeds = [
        _sc_gather_chunk(ids_flat, word_table, start, sch, seq_len, bsz)
        for start, sch in _CHUNKS
    ]
    o = None
    for (start, sch), e in zip(_CHUNKS, embeds):
        o = _tc_chunk(start, sch, n_tok, bsz, e, pos_table, wb, g2, b2, o)
    return o.reshape(bsz, seq_len, HID)


# speedup vs baseline: 5.2819x; 1.0484x over previous
import functools

import jax
import jax.numpy as jnp
from jax import lax
from jax.experimental import pallas as pl
from jax.experimental.pallas import tpu as pltpu
from jax.experimental.pallas import tpu_sc as plsc

EMB = 512
HID = 1024
EPS = 1e-07

_NC = 2
_NS = 16
_NW = _NC * _NS
_K = 4


def _sc_gather_body(ids_hbm, table_hbm, out_hbm, idx_v, buf_v, buf2_v, s0,
                    s1, *, k, sch, seq_len, bsz):
    tok_per_w = (sch * bsz) // _NW
    wpb = _NW // bsz
    wid = lax.axis_index("s") * _NC + lax.axis_index("c")
    myb = wid // wpb
    myj = wid % wpb
    half = tok_per_w // 2
    col = k * sch + myj * tok_per_w
    pltpu.sync_copy(ids_hbm.at[myb, pl.ds(col, tok_per_w)], idx_v)
    base = wid * tok_per_w
    g0 = pltpu.async_copy(table_hbm.at[idx_v.at[pl.ds(0, half)]], buf_v, s0)
    g1 = pltpu.async_copy(table_hbm.at[idx_v.at[pl.ds(half, half)]], buf2_v,
                          s1)
    g0.wait()
    pltpu.sync_copy(buf_v, out_hbm.at[pl.ds(base, half)])
    g1.wait()
    pltpu.sync_copy(buf2_v, out_hbm.at[pl.ds(base + half, half)])


def _sc_gather_chunk(ids2d, word_table, k, sch, seq_len, bsz):
    tok_per_w = (sch * bsz) // _NW
    mesh = plsc.VectorSubcoreMesh(core_axis_name="c", subcore_axis_name="s")
    body = functools.partial(_sc_gather_body, k=k, sch=sch, seq_len=seq_len,
                             bsz=bsz)
    kern = functools.partial(
        pl.kernel,
        mesh=mesh,
        out_type=jax.ShapeDtypeStruct((sch * bsz, EMB), jnp.float32),
        scratch_types=[
            pltpu.VMEM((tok_per_w,), jnp.int32),
            pltpu.VMEM((tok_per_w // 2, EMB), jnp.float32),
            pltpu.VMEM((tok_per_w // 2, EMB), jnp.float32),
            pltpu.SemaphoreType.DMA,
            pltpu.SemaphoreType.DMA,
        ],
    )(body)
    return kern(ids2d, word_table)


def _tc_body_first(x_ref, pos_ref, w_ref, g_ref, b_ref, o_ref):
    x = (x_ref[...] + pos_ref[...]).astype(jnp.bfloat16)
    h = jnp.dot(x, w_ref[...], preferred_element_type=jnp.float32)
    mu = jnp.mean(h, axis=-1, keepdims=True)
    var = jnp.mean((h - mu) ** 2, axis=-1, keepdims=True)
    o_ref[...] = (h - mu) * lax.rsqrt(var + EPS) * g_ref[...] + b_ref[...]


def _tc_body_chained(prev_ref, x_ref, pos_ref, w_ref, g_ref, b_ref, o_ref):
    del prev_ref
    _tc_body_first(x_ref, pos_ref, w_ref, g_ref, b_ref, o_ref)


def _tc_chunk(k, n_tok, nbatch, sch, embeds_k, pos_table, wb, g2, b2, prev):
    blk = sch
    blocks_per_batch = (n_tok // nbatch) // blk
    x_spec = pl.BlockSpec((blk, EMB), lambda j, b: (b, 0))
    pos_spec = pl.BlockSpec((blk, EMB), lambda j, b: (k, 0))
    w_spec = pl.BlockSpec((EMB, HID), lambda j, b: (0, 0))
    v_spec = pl.BlockSpec((1, HID), lambda j, b: (0, 0))
    out_spec = pl.BlockSpec(
        (blk, HID), lambda j, b: (b * blocks_per_batch + k, 0))
    out_shape = jax.ShapeDtypeStruct((n_tok, HID), jnp.float32)
    grid = (1, nbatch)
    if prev is None:
        return pl.pallas_call(
            _tc_body_first,
            grid=grid,
            in_specs=[x_spec, pos_spec, w_spec, v_spec, v_spec],
            out_specs=out_spec,
            out_shape=out_shape,
        )(embeds_k, pos_table, wb, g2, b2)
    return pl.pallas_call(
        _tc_body_chained,
        grid=grid,
        in_specs=[pl.BlockSpec(memory_space=pl.MemorySpace.ANY),
                  x_spec, pos_spec, w_spec, v_spec, v_spec],
        out_specs=out_spec,
        out_shape=out_shape,
        input_output_aliases={0: 0},
    )(prev, embeds_k, pos_table, wb, g2, b2)


def kernel(input_ids, word_table, pos_table, proj_w, ln_gamma, ln_beta):
    bsz, seq_len = input_ids.shape
    n_tok = bsz * seq_len
    ids2d = input_ids.astype(jnp.int32)
    sch = seq_len // _K
    wb = proj_w.astype(jnp.bfloat16)
    g2 = ln_gamma.reshape(1, HID)
    b2 = ln_beta.reshape(1, HID)
    embeds = [
        _sc_gather_chunk(ids2d, word_table, k, sch, seq_len, bsz)
        for k in range(_K)
    ]
    o = None
    for k in range(_K):
        o = _tc_chunk(k, n_tok, bsz, sch, embeds[k], pos_table, wb, g2, b2, o)
    return o.reshape(bsz, seq_len, HID)


# speedup vs baseline: 5.3890x; 1.0203x over previous
import functools

import jax
import jax.numpy as jnp
from jax import lax
from jax.experimental import pallas as pl
from jax.experimental.pallas import tpu as pltpu
from jax.experimental.pallas import tpu_sc as plsc

EMB = 512
HID = 1024
EPS = 1e-07

_NC = 2
_NS = 16
_NW = _NC * _NS
_K = 4


def _sc_gather_body(ids_hbm, table_hbm, out_hbm, idx_v, buf_v, sem, *, k,
                    sch, seq_len, bsz):
    tok_per_w = (sch * bsz) // _NW
    wpb = _NW // bsz
    wid = lax.axis_index("s") * _NC + lax.axis_index("c")
    myb = wid // wpb
    myj = wid % wpb
    col = k * sch + myj * tok_per_w
    pltpu.sync_copy(ids_hbm.at[myb, pl.ds(col, tok_per_w)], idx_v)
    pltpu.async_copy(table_hbm.at[idx_v], buf_v, sem).wait()
    pltpu.sync_copy(buf_v, out_hbm.at[pl.ds(wid * tok_per_w, tok_per_w)])


def _sc_gather_chunk(ids2d, word_table, k, sch, seq_len, bsz):
    tok_per_w = (sch * bsz) // _NW
    mesh = plsc.VectorSubcoreMesh(core_axis_name="c", subcore_axis_name="s")
    body = functools.partial(_sc_gather_body, k=k, sch=sch, seq_len=seq_len,
                             bsz=bsz)
    kern = functools.partial(
        pl.kernel,
        mesh=mesh,
        out_type=jax.ShapeDtypeStruct((sch * bsz, EMB), jnp.float32),
        scratch_types=[
            pltpu.VMEM((tok_per_w,), jnp.int32),
            pltpu.VMEM((tok_per_w, EMB), jnp.float32),
            pltpu.SemaphoreType.DMA,
        ],
    )(body)
    return kern(ids2d, word_table)


def _tc_body_first(x_ref, pos_ref, w_ref, g_ref, b_ref, o_ref):
    x = (x_ref[...] + pos_ref[...]).astype(jnp.bfloat16)
    h = jnp.dot(x, w_ref[...], preferred_element_type=jnp.float32)
    mu = jnp.mean(h, axis=-1, keepdims=True)
    var = jnp.mean((h - mu) ** 2, axis=-1, keepdims=True)
    o_ref[...] = (h - mu) * lax.rsqrt(var + EPS) * g_ref[...] + b_ref[...]


def _tc_body_chained(prev_ref, x_ref, pos_ref, w_ref, g_ref, b_ref, o_ref):
    del prev_ref
    _tc_body_first(x_ref, pos_ref, w_ref, g_ref, b_ref, o_ref)


def _tc_chunk(k, n_tok, nbatch, sch, embeds_k, pos_table, wb, g2, b2, prev):
    blk = sch
    blocks_per_batch = (n_tok // nbatch) // blk
    x_spec = pl.BlockSpec((blk, EMB), lambda j, b: (b, 0))
    pos_spec = pl.BlockSpec((blk, EMB), lambda j, b: (k, 0))
    w_spec = pl.BlockSpec((EMB, HID), lambda j, b: (0, 0))
    v_spec = pl.BlockSpec((1, HID), lambda j, b: (0, 0))
    out_spec = pl.BlockSpec(
        (blk, HID), lambda j, b: (b * blocks_per_batch + k, 0))
    out_shape = jax.ShapeDtypeStruct((n_tok, HID), jnp.float32)
    grid = (1, nbatch)
    if prev is None:
        return pl.pallas_call(
            _tc_body_first,
            grid=grid,
            in_specs=[x_spec, pos_spec, w_spec, v_spec, v_spec],
            out_specs=out_spec,
            out_shape=out_shape,
        )(embeds_k, pos_table, wb, g2, b2)
    return pl.pallas_call(
        _tc_body_chained,
        grid=grid,
        in_specs=[pl.BlockSpec(memory_space=pl.MemorySpace.ANY),
                  x_spec, pos_spec, w_spec, v_spec, v_spec],
        out_specs=out_spec,
        out_shape=out_shape,
        input_output_aliases={0: 0},
    )(prev, embeds_k, pos_table, wb, g2, b2)


def kernel(input_ids, word_table, pos_table, proj_w, ln_gamma, ln_beta):
    bsz, seq_len = input_ids.shape
    n_tok = bsz * seq_len
    ids2d = input_ids.astype(jnp.int32)
    sch = seq_len // _K
    wb = proj_w.astype(jnp.bfloat16)
    g2 = ln_gamma.reshape(1, HID)
    b2 = ln_beta.reshape(1, HID)
    embeds = [
        _sc_gather_chunk(ids2d, word_table, k, sch, seq_len, bsz)
        for k in range(_K)
    ]
    o = None
    for k in range(_K):
        o = _tc_chunk(k, n_tok, bsz, sch, embeds[k], pos_table, wb, g2, b2, o)
    return o.reshape(bsz, seq_len, HID)
